# baseline (speedup 1.0000x reference)
import functools

import jax
import jax.numpy as jnp
from jax import lax
from jax.experimental import pallas as pl
from jax.experimental.pallas import tpu as pltpu
from jax.experimental.pallas import tpu_sc as plsc

_VOCAB = 100000
_D = 64
_B = 1024
_HIST = 50
_HIST_PAD = 64

_NC, _NS = 2, 16
_NW = _NC * _NS
_ROWS_PER_W = _B // _NW
_PAIRS = _ROWS_PER_W // 2
_LANES = 16
_DCH = _D // _LANES

_mesh = plsc.VectorSubcoreMesh(core_axis_name="c", subcore_axis_name="s")


@functools.partial(
    pl.kernel,
    out_type=jax.ShapeDtypeStruct((_B, _D), jnp.float32),
    mesh=_mesh,
    scratch_types=[
        pltpu.VMEM((_PAIRS, 2 * _HIST_PAD), jnp.int32),
        pltpu.VMEM((2 * _HIST_PAD, _D), jnp.float32),
        pltpu.VMEM((_ROWS_PER_W, _D), jnp.float32),
        pltpu.SemaphoreType.DMA,
    ],
    compiler_params=pltpu.CompilerParams(use_tc_tiling_on_sc=False),
)
def _sc_pool(idx_hbm, table_hbm, x_hbm, idx_v, rows_v, xout_v, sem):
    wid = lax.axis_index("s") * _NC + lax.axis_index("c")
    base = wid * _ROWS_PER_W
    pltpu.sync_copy(idx_hbm.at[wid], idx_v)

    def pair_body(p, carry):
        pltpu.async_copy(table_hbm.at[idx_v.at[p]], rows_v, sem).wait()
        for half in range(2):
            accs = [jnp.zeros((_LANES,), jnp.float32) for _ in range(_DCH)]
            for j in range(_HIST):
                for k in range(_DCH):
                    accs[k] = accs[k] + rows_v[half * _HIST_PAD + j,
                                               pl.ds(k * _LANES, _LANES)]
            for k in range(_DCH):
                xout_v[2 * p + half, pl.ds(k * _LANES, _LANES)] = (
                    accs[k] * (1.0 / _HIST))
        return carry

    lax.fori_loop(0, _PAIRS, pair_body, 0)
    pltpu.sync_copy(xout_v, x_hbm.at[pl.ds(base, _ROWS_PER_W)])


_VT = 2048


def _mm_body(x_ref, w_ref, b_ref, o_ref):
    o_ref[...] = lax.dot_general(
        x_ref[...], w_ref[...],
        dimension_numbers=(((1,), (1,)), ((), ())),
        preferred_element_type=jnp.float32,
    ) + b_ref[...]


def _project(x, W, b2d):
    grid = pl.cdiv(_VOCAB, _VT)
    return pl.pallas_call(
        _mm_body,
        grid=(grid,),
        in_specs=[
            pl.BlockSpec((_B, _D), lambda i: (0, 0)),
            pl.BlockSpec((_VT, _D), lambda i: (i, 0)),
            pl.BlockSpec((1, _VT), lambda i: (0, i)),
        ],
        out_specs=pl.BlockSpec((_B, _VT), lambda i: (0, i)),
        out_shape=jax.ShapeDtypeStruct((_B, _VOCAB), jnp.float32),
    )(x, W, b2d)


def kernel(inputs, table, W, b):
    idx3 = jnp.pad(inputs, ((0, 0), (0, _HIST_PAD - _HIST))).reshape(
        _NW, _PAIRS, 2 * _HIST_PAD)
    x = _sc_pool(idx3, table)
    return _project(x, W, b.reshape(1, _VOCAB))
